# baseline (speedup 1.0000x reference)
import functools

import jax
import jax.numpy as jnp
from jax import lax
from jax.experimental import pallas as pl
from jax.experimental.pallas import tpu as pltpu
from jax.experimental.pallas import tpu_sc as plsc

_INFO = plsc.get_sparse_core_info()
_NC = _INFO.num_cores
_NS = _INFO.num_subcores
_NW = _NC * _NS

_B = 128 * 1024
_D = 64
_DP = 128
_C = 128
_NCHUNK = _B // _C
_CPW = _NCHUNK // _NW
_K = 2
_ROWS = _K * _C
_NBLK = _CPW // _K


def _gather_body(codebook_hbm, idx_hbm, out_hbm,
                 idx_v, rows_ga, rows_gb, pbuf_a, pbuf_b,
                 gsem_a, gsem_b, wsem_a, wsem_b):
    wid = lax.axis_index("s") * _NC + lax.axis_index("c")
    row0 = wid * _CPW
    pltpu.sync_copy(idx_hbm.at[pl.ds(row0, _CPW)], idx_v)

    def fire_gathers(j, buf, gsem):
        for k in range(_K):
            pltpu.async_copy(
                codebook_hbm.at[idx_v.at[j * _K + k]],
                buf.at[pl.ds(k * _C, _C)],
                gsem,
            )

    def wait_gathers(buf, gsem):
        for k in range(_K):
            pltpu.make_async_copy(
                codebook_hbm.at[idx_v.at[0]],
                buf.at[pl.ds(k * _C, _C)],
                gsem,
            ).wait()

    def wait_write(pbuf, wsem):
        pltpu.make_async_copy(
            pbuf, out_hbm.at[0, :, pl.ds(0, _ROWS)], wsem).wait()

    def transpose(buf, pbuf):
        def grp(g, carry):
            rows = lax.iota(jnp.int32, 16) + g * 16
            for d in range(_D):
                col = jnp.full((16,), d, jnp.int32)
                vals = plsc.load_gather(buf, [rows, col])
                pbuf[d, pl.ds(g * 16, 16)] = vals
            return carry
        lax.fori_loop(0, _ROWS // 16, grp, 0)

    def fire_write(j, pbuf, wsem):
        flat0 = (row0 + j * _K) * _C
        pltpu.async_copy(
            pbuf,
            out_hbm.at[flat0 // 1024, :, pl.ds(flat0 % 1024, _ROWS)],
            wsem,
        )

    def handle(p, j, buf, gsem, pbuf, wsem, next_j, next_buf, next_gsem,
               guard_next):
        if guard_next:
            @pl.when(next_j < _NBLK)
            def _():
                fire_gathers(next_j, next_buf, next_gsem)
        else:
            fire_gathers(next_j, next_buf, next_gsem)
        wait_gathers(buf, gsem)

        @pl.when(p > 0)
        def _():
            wait_write(pbuf, wsem)
        transpose(buf, pbuf)
        fire_write(j, pbuf, wsem)

    fire_gathers(0, rows_ga, gsem_a)

    def pair(p, carry):
        handle(p, 2 * p, rows_ga, gsem_a, pbuf_a, wsem_a,
               2 * p + 1, rows_gb, gsem_b, False)
        handle(p, 2 * p + 1, rows_gb, gsem_b, pbuf_b, wsem_b,
               2 * p + 2, rows_ga, gsem_a, True)
        return carry

    lax.fori_loop(0, _NBLK // 2, pair, 0)
    wait_write(pbuf_a, wsem_a)
    wait_write(pbuf_b, wsem_b)


@functools.partial(jax.jit, static_argnames=())
def _gather(codebook_p, idx2d):
    k = pl.kernel(
        _gather_body,
        out_type=jax.ShapeDtypeStruct((128, _D, 1024), jnp.float32),
        mesh=plsc.VectorSubcoreMesh(core_axis_name="c", subcore_axis_name="s"),
        scratch_types=[
            pltpu.VMEM((_CPW, _C), jnp.int32),
            pltpu.VMEM((_ROWS, _DP), jnp.float32),
            pltpu.VMEM((_ROWS, _DP), jnp.float32),
            pltpu.VMEM((_D, _ROWS), jnp.float32),
            pltpu.VMEM((_D, _ROWS), jnp.float32),
            pltpu.SemaphoreType.DMA,
            pltpu.SemaphoreType.DMA,
            pltpu.SemaphoreType.DMA,
            pltpu.SemaphoreType.DMA,
        ],
        compiler_params=pltpu.CompilerParams(
            use_tc_tiling_on_sc=True, needs_layout_passes=False),
    )
    return k(codebook_p, idx2d)


def kernel(indices, codebook):
    idx2d = indices.reshape(_NCHUNK, _C)
    codebook_p = jnp.pad(codebook, ((0, 0), (0, _DP - _D)))
    return _gather(codebook_p, idx2d).transpose(0, 2, 1)


# speedup vs baseline: 1.2034x; 1.2034x over previous
import functools

import jax
import jax.numpy as jnp
from jax import lax
from jax.experimental import pallas as pl
from jax.experimental.pallas import tpu as pltpu
from jax.experimental.pallas import tpu_sc as plsc

_INFO = plsc.get_sparse_core_info()
_NC = _INFO.num_cores
_NS = _INFO.num_subcores
_NW = _NC * _NS

_B = 128 * 1024
_D = 64
_DP = 128
_C = 128
_NCHUNK = _B // _C
_CPW = _NCHUNK // _NW
_K = 2
_ROWS = _K * _C
_NBLK = _CPW // _K


def _gather_body(codebook_hbm, idx_hbm, out_hbm,
                 idx_v, rows_ga, rows_gb, pbuf_a, pbuf_b,
                 gsem_a, gsem_b, wsem_a, wsem_b):
    wid = lax.axis_index("s") * _NC + lax.axis_index("c")
    row0 = wid * _CPW
    pltpu.sync_copy(idx_hbm.at[pl.ds(row0, _CPW)], idx_v)

    def fire_gathers(j, buf, gsem):
        for k in range(_K):
            pltpu.async_copy(
                codebook_hbm.at[idx_v.at[j * _K + k]],
                buf.at[pl.ds(k * _C, _C)],
                gsem,
            )

    def wait_gathers(buf, gsem):
        for k in range(_K):
            pltpu.make_async_copy(
                codebook_hbm.at[idx_v.at[0]],
                buf.at[pl.ds(k * _C, _C)],
                gsem,
            ).wait()

    def wait_write(pbuf, wsem):
        pltpu.make_async_copy(
            pbuf, out_hbm.at[0, :, pl.ds(0, _ROWS)], wsem).wait()

    def transpose(buf, pbuf):
        dvecs = [lax.iota(jnp.int32, 16) + 16 * k for k in range(_D // 16)]

        def rowstep(r, carry):
            for u in range(4):
                col = jnp.full((16,), r * 4 + u, jnp.int32)
                for k in range(_D // 16):
                    vals = buf[r * 4 + u, pl.ds(k * 16, 16)]
                    plsc.store_scatter(pbuf, [dvecs[k], col], vals)
            return carry

        lax.fori_loop(0, _ROWS // 4, rowstep, 0)

    def fire_write(j, pbuf, wsem):
        flat0 = (row0 + j * _K) * _C
        pltpu.async_copy(
            pbuf,
            out_hbm.at[flat0 // 1024, :, pl.ds(flat0 % 1024, _ROWS)],
            wsem,
        )

    def handle(p, j, buf, gsem, pbuf, wsem, next_j, next_buf, next_gsem,
               guard_next):
        if guard_next:
            @pl.when(next_j < _NBLK)
            def _():
                fire_gathers(next_j, next_buf, next_gsem)
        else:
            fire_gathers(next_j, next_buf, next_gsem)
        wait_gathers(buf, gsem)

        @pl.when(p > 0)
        def _():
            wait_write(pbuf, wsem)
        transpose(buf, pbuf)
        fire_write(j, pbuf, wsem)

    fire_gathers(0, rows_ga, gsem_a)

    def pair(p, carry):
        handle(p, 2 * p, rows_ga, gsem_a, pbuf_a, wsem_a,
               2 * p + 1, rows_gb, gsem_b, False)
        handle(p, 2 * p + 1, rows_gb, gsem_b, pbuf_b, wsem_b,
               2 * p + 2, rows_ga, gsem_a, True)
        return carry

    lax.fori_loop(0, _NBLK // 2, pair, 0)
    wait_write(pbuf_a, wsem_a)
    wait_write(pbuf_b, wsem_b)


@functools.partial(jax.jit, static_argnames=())
def _gather(codebook_p, idx2d):
    k = pl.kernel(
        _gather_body,
        out_type=jax.ShapeDtypeStruct((128, _D, 1024), jnp.float32),
        mesh=plsc.VectorSubcoreMesh(core_axis_name="c", subcore_axis_name="s"),
        scratch_types=[
            pltpu.VMEM((_CPW, _C), jnp.int32),
            pltpu.VMEM((_ROWS, _DP), jnp.float32),
            pltpu.VMEM((_ROWS, _DP), jnp.float32),
            pltpu.VMEM((_D, _ROWS), jnp.float32),
            pltpu.VMEM((_D, _ROWS), jnp.float32),
            pltpu.SemaphoreType.DMA,
            pltpu.SemaphoreType.DMA,
            pltpu.SemaphoreType.DMA,
            pltpu.SemaphoreType.DMA,
        ],
        compiler_params=pltpu.CompilerParams(
            use_tc_tiling_on_sc=True, needs_layout_passes=False,
            disable_bounds_checks=True),
    )
    return k(codebook_p, idx2d)


def kernel(indices, codebook):
    idx2d = indices.reshape(_NCHUNK, _C)
    codebook_p = jnp.pad(codebook, ((0, 0), (0, _DP - _D)))
    return _gather(codebook_p, idx2d).transpose(0, 2, 1)


# speedup vs baseline: 1.8769x; 1.5597x over previous
import functools

import jax
import jax.numpy as jnp
from jax import lax
from jax.experimental import pallas as pl
from jax.experimental.pallas import tpu as pltpu
from jax.experimental.pallas import tpu_sc as plsc

_INFO = plsc.get_sparse_core_info()
_NC = _INFO.num_cores
_NS = _INFO.num_subcores
_NW = _NC * _NS

_B = 128 * 1024
_D = 64
_DP = 128
_C = 128
_NCHUNK = _B // _C
_CPW = _NCHUNK // _NW
_K = 2
_ROWS = _K * _C
_G = _CPW // _K


def _gather_body(codebook_hbm, idx_hbm, out_hbm,
                 idx_v, rows_ga, rows_gb, rows_c, gsem_a, gsem_b, wsem):
    wid = lax.axis_index("s") * _NC + lax.axis_index("c")
    row0 = wid * _CPW
    pltpu.sync_copy(idx_hbm.at[pl.ds(row0, _CPW)], idx_v)

    def fire_gathers(g, buf, gsem):
        for k in range(_K):
            pltpu.async_copy(
                codebook_hbm.at[idx_v.at[g * _K + k]],
                buf.at[pl.ds(k * _C, _C)],
                gsem,
            )

    def wait_gathers(buf, gsem):
        for k in range(_K):
            pltpu.make_async_copy(
                codebook_hbm.at[idx_v.at[0]],
                buf.at[pl.ds(k * _C, _C)],
                gsem,
            ).wait()

    def wait_write():
        pltpu.make_async_copy(
            rows_c, out_hbm.at[0, pl.ds(0, _ROWS)], wsem).wait()

    def compact(buf):
        def rowstep(r, carry):
            for u in range(4):
                for k in range(_D // 16):
                    rows_c[r * 4 + u, pl.ds(k * 16, 16)] = (
                        buf[r * 4 + u, pl.ds(k * 16, 16)])
            return carry
        lax.fori_loop(0, _ROWS // 4, rowstep, 0)

    def fire_write(g):
        flat0 = (row0 + g * _K) * _C
        pltpu.async_copy(
            rows_c,
            out_hbm.at[flat0 // 1024, pl.ds(flat0 % 1024, _ROWS)],
            wsem,
        )

    def handle(p, g, buf, gsem, next_g, next_buf, next_gsem, guard_next):
        if guard_next:
            @pl.when(next_g < _G)
            def _():
                fire_gathers(next_g, next_buf, next_gsem)
        else:
            fire_gathers(next_g, next_buf, next_gsem)
        wait_gathers(buf, gsem)

        @pl.when(p > 0)
        def _():
            wait_write()
        compact(buf)
        fire_write(g)

    fire_gathers(0, rows_ga, gsem_a)

    def pair(p, carry):
        handle(p, 2 * p, rows_ga, gsem_a, 2 * p + 1, rows_gb, gsem_b, False)
        handle(p + 1, 2 * p + 1, rows_gb, gsem_b,
               2 * p + 2, rows_ga, gsem_a, True)
        return carry

    lax.fori_loop(0, _G // 2, pair, 0)
    wait_write()


@functools.partial(jax.jit, static_argnames=())
def _gather(codebook_p, idx2d):
    k = pl.kernel(
        _gather_body,
        out_type=jax.ShapeDtypeStruct((128, 1024, _D), jnp.float32),
        mesh=plsc.VectorSubcoreMesh(core_axis_name="c", subcore_axis_name="s"),
        scratch_types=[
            pltpu.VMEM((_CPW, _C), jnp.int32),
            pltpu.VMEM((_ROWS, _DP), jnp.float32),
            pltpu.VMEM((_ROWS, _DP), jnp.float32),
            pltpu.VMEM((_ROWS, _D), jnp.float32),
            pltpu.SemaphoreType.DMA,
            pltpu.SemaphoreType.DMA,
            pltpu.SemaphoreType.DMA,
        ],
        compiler_params=pltpu.CompilerParams(
            use_tc_tiling_on_sc=True, needs_layout_passes=False,
            disable_bounds_checks=True),
    )
    return k(codebook_p, idx2d)


def kernel(indices, codebook):
    idx2d = indices.reshape(_NCHUNK, _C)
    codebook_p = jnp.pad(codebook, ((0, 0), (0, _DP - _D)))
    return _gather(codebook_p, idx2d)
